# Initial kernel scaffold; baseline (speedup 1.0000x reference)
import jax
import jax.numpy as jnp
from jax.experimental import pallas as pl


def kernel(xs, ys, x):
    raise NotImplementedError("write your pallas kernel here")


# speedup vs baseline: 7.4366x; 7.4366x over previous
import dataclasses

import jax
import jax.numpy as jnp
from jax import lax
from jax.experimental import pallas as pl
from jax.experimental.pallas import tpu as pltpu
from jax.experimental.pallas import tpu_sc as plsc

K = 100000
D = 128
Q = 16384

NC = 2
NS = 16
L = 16
NW = NC * NS
QPW = Q // NW
CHUNK = 64
NCHUNK = QPW // CHUNK
SEARCH_STEPS = 17


def _sc_kernel(xs_hbm, ys_hbm, x_hbm, out_hbm,
               xs_v, x_v, idx_v, buf0, buf1, sem_in, sem_g0, sem_g1):
    wid = lax.axis_index("s") * NC + lax.axis_index("c")
    base = wid * QPW

    h_xs = pltpu.async_copy(xs_hbm, xs_v, sem_in)
    h_x = pltpu.async_copy(x_hbm.at[pl.ds(base, QPW)], x_v, sem_g0)
    h_x.wait()
    h_xs.wait()

    @pl.loop(0, QPW, step=L)
    def _(i):
        xq = x_v[pl.ds(i, L)]
        lo = jnp.zeros((L,), jnp.int32)
        hi = jnp.full((L,), K, jnp.int32)
        for _step in range(SEARCH_STEPS):
            mid = jnp.right_shift(lo + hi, 1)
            mid_safe = jnp.minimum(mid, K - 1)
            xv = plsc.load_gather(xs_v, [mid_safe])
            valid = lo < hi
            le = xv <= xq
            lo = jnp.where(valid & le, mid + 1, lo)
            hi = jnp.where(valid & jnp.logical_not(le), mid, hi)
        row = jnp.where(lo == 0, K - 1, lo - 1)
        idx_v[pl.ds(i, L)] = row

    bufs = (buf0, buf1)
    sems = (sem_g0, sem_g1)

    def start(c):
        return pltpu.async_copy(
            ys_hbm.at[idx_v.at[pl.ds(c * CHUNK, CHUNK)]], bufs[c % 2], sems[c % 2])

    h_next = start(0)
    for c in range(NCHUNK):
        h = h_next
        if c + 1 < NCHUNK:
            h_next = start(c + 1)
        h.wait()
        pltpu.sync_copy(bufs[c % 2], out_hbm.at[pl.ds(base + c * CHUNK, CHUNK)])


def kernel(xs, ys, x):
    mesh = plsc.VectorSubcoreMesh(core_axis_name="c", subcore_axis_name="s")
    cp = pltpu.CompilerParams()
    if "needs_layout_passes" in pltpu.CompilerParams.__dataclass_fields__:
        cp = dataclasses.replace(cp, needs_layout_passes=False)
    run = pl.kernel(
        _sc_kernel,
        out_type=jax.ShapeDtypeStruct((Q, D), jnp.float32),
        mesh=mesh,
        scratch_types=[
            pltpu.VMEM((K,), jnp.float32),
            pltpu.VMEM((QPW,), jnp.float32),
            pltpu.VMEM((QPW,), jnp.int32),
            pltpu.VMEM((CHUNK, D), jnp.float32),
            pltpu.VMEM((CHUNK, D), jnp.float32),
            pltpu.SemaphoreType.DMA,
            pltpu.SemaphoreType.DMA,
            pltpu.SemaphoreType.DMA,
        ],
        compiler_params=cp,
    )
    return run(xs, ys, x)


# speedup vs baseline: 7.8820x; 1.0599x over previous
import dataclasses

import jax
import jax.numpy as jnp
from jax import lax
from jax.experimental import pallas as pl
from jax.experimental.pallas import tpu as pltpu
from jax.experimental.pallas import tpu_sc as plsc

K = 100000
D = 128
Q = 16384

NC = 2
NS = 16
L = 16
NW = NC * NS
QPW = Q // NW
CHUNK = 64
NCHUNK = QPW // CHUNK
SEARCH_STEPS = 17


def _sc_kernel(xs_hbm, ys_hbm, x_hbm, out_hbm,
               xs_v, x_v, idx_v, buf0, buf1, sem_in, sem_g0, sem_g1):
    wid = lax.axis_index("s") * NC + lax.axis_index("c")
    base = wid * QPW

    h_xs = pltpu.async_copy(xs_hbm, xs_v, sem_in)
    h_x = pltpu.async_copy(x_hbm.at[pl.ds(base, QPW)], x_v, sem_g0)
    h_x.wait()
    h_xs.wait()

    @plsc.parallel_loop(0, QPW, step=L, unroll=4)
    def _(i):
        xq = x_v[pl.ds(i, L)]
        lo = jnp.zeros((L,), jnp.int32)
        hi = jnp.full((L,), K, jnp.int32)
        for _step in range(SEARCH_STEPS):
            mid = jnp.right_shift(lo + hi, 1)
            mid_safe = jnp.minimum(mid, K - 1)
            xv = plsc.load_gather(xs_v, [mid_safe])
            valid = lo < hi
            le = xv <= xq
            lo = jnp.where(valid & le, mid + 1, lo)
            hi = jnp.where(valid & jnp.logical_not(le), mid, hi)
        row = jnp.where(lo == 0, K - 1, lo - 1)
        idx_v[pl.ds(i, L)] = row

    bufs = (buf0, buf1)
    sems = (sem_g0, sem_g1)

    def start(c):
        return pltpu.async_copy(
            ys_hbm.at[idx_v.at[pl.ds(c * CHUNK, CHUNK)]], bufs[c % 2], sems[c % 2])

    h_next = start(0)
    for c in range(NCHUNK):
        h = h_next
        if c + 1 < NCHUNK:
            h_next = start(c + 1)
        h.wait()
        pltpu.sync_copy(bufs[c % 2], out_hbm.at[pl.ds(base + c * CHUNK, CHUNK)])


def kernel(xs, ys, x):
    mesh = plsc.VectorSubcoreMesh(core_axis_name="c", subcore_axis_name="s")
    cp = pltpu.CompilerParams()
    if "needs_layout_passes" in pltpu.CompilerParams.__dataclass_fields__:
        cp = dataclasses.replace(cp, needs_layout_passes=False)
    run = pl.kernel(
        _sc_kernel,
        out_type=jax.ShapeDtypeStruct((Q, D), jnp.float32),
        mesh=mesh,
        scratch_types=[
            pltpu.VMEM((K,), jnp.float32),
            pltpu.VMEM((QPW,), jnp.float32),
            pltpu.VMEM((QPW,), jnp.int32),
            pltpu.VMEM((CHUNK, D), jnp.float32),
            pltpu.VMEM((CHUNK, D), jnp.float32),
            pltpu.SemaphoreType.DMA,
            pltpu.SemaphoreType.DMA,
            pltpu.SemaphoreType.DMA,
        ],
        compiler_params=cp,
    )
    return run(xs, ys, x)
